# Initial kernel scaffold; baseline (speedup 1.0000x reference)
import jax
import jax.numpy as jnp
from jax.experimental import pallas as pl


def kernel(x, edge_index, W1, b1, lW1, lb1, lW2, lb2, lW3, lb3):
    raise NotImplementedError("write your pallas kernel here")


# speedup vs baseline: 22.2139x; 22.2139x over previous
import functools

import jax
import jax.numpy as jnp
from jax import lax
from jax.experimental import pallas as pl
from jax.experimental.pallas import tpu as pltpu
from jax.experimental.pallas import tpu_sc as plsc

N = 10000
E = 320000
D = 128
H1, H2, H3 = 64, 32, 16
NCLS = 10

NSC = 2
NSUB = 16
NW = NSC * NSUB
CHUNK = 128
K = 80
E_PAD = NW * K * CHUNK
N_PAD = 10240
ROWS_PER_SUB = N_PAD // NSUB
DEG_W = 16

_mesh = plsc.VectorSubcoreMesh(core_axis_name="c", subcore_axis_name="s")
_sc_params = pltpu.CompilerParams(use_tc_tiling_on_sc=False)


def _deg_body(dst_hbm, out_hbm, idx_v, val_v, zbuf_v, deg_sh):
    cid = lax.axis_index("c")
    sid = lax.axis_index("s")
    wid = sid * NSC + cid

    @pl.loop(0, ROWS_PER_SUB)
    def _(r):
        zbuf_v[r, :] = jnp.zeros((DEG_W,), jnp.float32)

    pltpu.sync_copy(zbuf_v, deg_sh.at[pl.ds(sid * ROWS_PER_SUB, ROWS_PER_SUB)])

    @pl.loop(0, CHUNK)
    def _(r):
        val_v[r, :] = jnp.ones((DEG_W,), jnp.float32)

    pltpu.sync_copy(dst_hbm.at[wid], idx_v)
    plsc.subcore_barrier()

    @pl.loop(0, K)
    def _(j):
        pltpu.sync_copy(val_v, deg_sh.at[idx_v.at[j]], add=True)

    plsc.subcore_barrier()

    sl = pl.ds(sid * ROWS_PER_SUB, ROWS_PER_SUB)
    pltpu.sync_copy(deg_sh.at[sl], zbuf_v)
    pltpu.sync_copy(zbuf_v, out_hbm.at[cid].at[sl])


def _deg_partials(dst_r):
    return pl.kernel(
        _deg_body,
        out_type=jax.ShapeDtypeStruct((NSC, N_PAD, DEG_W), jnp.float32),
        mesh=_mesh,
        scratch_types=[
            pltpu.VMEM((K, CHUNK), jnp.int32),
            pltpu.VMEM((CHUNK, DEG_W), jnp.float32),
            pltpu.VMEM((ROWS_PER_SUB, DEG_W), jnp.float32),
            pltpu.VMEM_SHARED((N_PAD, DEG_W), jnp.float32),
        ],
        compiler_params=_sc_params,
    )(dst_r)


def _agg_body(src_hbm, dst_hbm, y_hbm, out_hbm,
              srcv, dstv, rows0, rows1, zbuf_v, agg_sh, sem0, sem1):
    cid = lax.axis_index("c")
    sid = lax.axis_index("s")
    wid = sid * NSC + cid

    @pl.loop(0, ROWS_PER_SUB)
    def _(r):
        for h in range(H1 // 16):
            zbuf_v[r, pl.ds(h * 16, 16)] = jnp.zeros((16,), jnp.float32)

    pltpu.sync_copy(zbuf_v, agg_sh.at[pl.ds(sid * ROWS_PER_SUB, ROWS_PER_SUB)])

    pltpu.sync_copy(src_hbm.at[wid], srcv)
    pltpu.sync_copy(dst_hbm.at[wid], dstv)
    plsc.subcore_barrier()

    @pl.loop(0, K, step=2)
    def _(j):
        cp0 = pltpu.async_copy(y_hbm.at[srcv.at[j]], rows0, sem0)
        cp1 = pltpu.async_copy(y_hbm.at[srcv.at[j + 1]], rows1, sem1)
        cp0.wait()
        pltpu.sync_copy(rows0, agg_sh.at[dstv.at[j]], add=True)
        cp1.wait()
        pltpu.sync_copy(rows1, agg_sh.at[dstv.at[j + 1]], add=True)

    plsc.subcore_barrier()

    sl = pl.ds(sid * ROWS_PER_SUB, ROWS_PER_SUB)
    pltpu.sync_copy(agg_sh.at[sl], zbuf_v)
    pltpu.sync_copy(zbuf_v, out_hbm.at[cid].at[sl])


def _agg_partials(src_r, dst_r, y):
    return pl.kernel(
        _agg_body,
        out_type=jax.ShapeDtypeStruct((NSC, N_PAD, H1), jnp.float32),
        mesh=_mesh,
        scratch_types=[
            pltpu.VMEM((K, CHUNK), jnp.int32),
            pltpu.VMEM((K, CHUNK), jnp.int32),
            pltpu.VMEM((CHUNK, H1), jnp.float32),
            pltpu.VMEM((CHUNK, H1), jnp.float32),
            pltpu.VMEM((ROWS_PER_SUB, H1), jnp.float32),
            pltpu.VMEM_SHARED((N_PAD, H1), jnp.float32),
            pltpu.SemaphoreType.DMA,
            pltpu.SemaphoreType.DMA,
        ],
        compiler_params=_sc_params,
    )(src_r, dst_r, y)


def _xw_body(x_ref, w_ref, o_ref):
    o_ref[...] = jnp.dot(x_ref[...], w_ref[...],
                         preferred_element_type=jnp.float32)


def _scale_body(xw_ref, dp_ref, y_ref, dinv_ref):
    deg = dp_ref[0] + dp_ref[1] + 1.0
    dinv = lax.rsqrt(deg)
    dinv_ref[...] = dinv
    y_ref[...] = xw_ref[...] * dinv


def _mlp_body(ap_ref, y_ref, dinv_ref, b1_ref, lw1_ref, lb1_ref,
              lw2_ref, lb2_ref, lw3_ref, lb3_ref, o_ref):
    agg = ap_ref[0] + ap_ref[1] + y_ref[...]
    h = jnp.maximum(agg * dinv_ref[...] + b1_ref[...], 0.0)
    h = jnp.maximum(jnp.dot(h, lw1_ref[...],
                            preferred_element_type=jnp.float32) + lb1_ref[...], 0.0)
    h = jnp.maximum(jnp.dot(h, lw2_ref[...],
                            preferred_element_type=jnp.float32) + lb2_ref[...], 0.0)
    o_ref[...] = jnp.dot(h, lw3_ref[...],
                         preferred_element_type=jnp.float32) + lb3_ref[...]


def kernel(x, edge_index, W1, b1, lW1, lb1, lW2, lb2, lW3, lb3):
    f32 = jnp.float32
    src = edge_index[0]
    dst = edge_index[1]
    pad = E_PAD - E
    src_r = jnp.concatenate(
        [src, jnp.full((pad,), N, jnp.int32)]).reshape(NW, K, CHUNK)
    dst_r = jnp.concatenate(
        [dst, jnp.full((pad,), N, jnp.int32)]).reshape(NW, K, CHUNK)
    x_pad = jnp.pad(x, ((0, N_PAD - N), (0, 0)))

    deg_p = _deg_partials(dst_r)
    dp = deg_p[:, :, 0:1]

    xw = pl.pallas_call(
        _xw_body,
        out_shape=jax.ShapeDtypeStruct((N_PAD, H1), f32),
    )(x_pad, W1)

    y, dinv = pl.pallas_call(
        _scale_body,
        out_shape=[jax.ShapeDtypeStruct((N_PAD, H1), f32),
                   jax.ShapeDtypeStruct((N_PAD, 1), f32)],
    )(xw, dp)

    agg_p = _agg_partials(src_r, dst_r, y)

    out = pl.pallas_call(
        _mlp_body,
        out_shape=jax.ShapeDtypeStruct((N_PAD, NCLS), f32),
    )(agg_p, y, dinv, b1.reshape(1, H1), lW1, lb1.reshape(1, H2),
      lW2, lb2.reshape(1, H3), lW3, lb3.reshape(1, NCLS))

    return out[:N]


# speedup vs baseline: 27.0580x; 1.2181x over previous
import functools

import jax
import jax.numpy as jnp
from jax import lax
from jax.experimental import pallas as pl
from jax.experimental.pallas import tpu as pltpu
from jax.experimental.pallas import tpu_sc as plsc

N = 10000
E = 320000
D = 128
H1, H2, H3 = 64, 32, 16
NCLS = 10

NSC = 2
NSUB = 16
NW = NSC * NSUB
CHUNK = 128
K = 80
E_PAD = NW * K * CHUNK
N_PAD = 10240
ROWS_PER_SUB = N_PAD // NSUB
DEG_W = 16

_mesh = plsc.VectorSubcoreMesh(core_axis_name="c", subcore_axis_name="s")
_sc_params = pltpu.CompilerParams(use_tc_tiling_on_sc=False)


def _deg_body(dst_hbm, out_hbm, idx_v, val_v, zbuf_v, deg_sh, sem0):
    cid = lax.axis_index("c")
    sid = lax.axis_index("s")
    wid = sid * NSC + cid

    @pl.loop(0, ROWS_PER_SUB)
    def _(r):
        zbuf_v[r, :] = jnp.zeros((DEG_W,), jnp.float32)

    pltpu.sync_copy(zbuf_v, deg_sh.at[pl.ds(sid * ROWS_PER_SUB, ROWS_PER_SUB)])

    @pl.loop(0, CHUNK)
    def _(r):
        val_v[r, :] = jnp.ones((DEG_W,), jnp.float32)

    pltpu.sync_copy(dst_hbm.at[wid], idx_v)
    plsc.subcore_barrier()

    @pl.loop(0, K, step=8)
    def _(j):
        cps = [pltpu.async_copy(val_v, deg_sh.at[idx_v.at[j + b]], sem0,
                                add=True)
               for b in range(8)]
        for cp in cps:
            cp.wait()

    plsc.subcore_barrier()

    sl = pl.ds(sid * ROWS_PER_SUB, ROWS_PER_SUB)
    pltpu.sync_copy(deg_sh.at[sl], zbuf_v)
    pltpu.sync_copy(zbuf_v, out_hbm.at[cid].at[sl])


def _deg_partials(dst_r):
    return pl.kernel(
        _deg_body,
        out_type=jax.ShapeDtypeStruct((NSC, N_PAD, DEG_W), jnp.float32),
        mesh=_mesh,
        scratch_types=[
            pltpu.VMEM((K, CHUNK), jnp.int32),
            pltpu.VMEM((CHUNK, DEG_W), jnp.float32),
            pltpu.VMEM((ROWS_PER_SUB, DEG_W), jnp.float32),
            pltpu.VMEM_SHARED((N_PAD, DEG_W), jnp.float32),
            pltpu.SemaphoreType.DMA,
        ],
        compiler_params=_sc_params,
    )(dst_r)


NB = 8
NZP = ROWS_PER_SUB // CHUNK


def _agg_body(src_hbm, dst_hbm, y_hbm, out_hbm,
              srcv, dstv, rows_v, agg_sh, gsem, ssem):
    cid = lax.axis_index("c")
    sid = lax.axis_index("s")
    wid = sid * NSC + cid

    @pl.loop(0, CHUNK)
    def _(r):
        for h in range(H1 // 16):
            rows_v[0, r, pl.ds(h * 16, 16)] = jnp.zeros((16,), jnp.float32)

    for p in range(NZP):
        pltpu.sync_copy(
            rows_v.at[0],
            agg_sh.at[pl.ds(sid * ROWS_PER_SUB + p * CHUNK, CHUNK)])

    pltpu.sync_copy(src_hbm.at[wid], srcv)
    pltpu.sync_copy(dst_hbm.at[wid], dstv)
    plsc.subcore_barrier()

    def _gather(c, b):
        return pltpu.async_copy(y_hbm.at[srcv.at[c]], rows_v.at[b],
                                gsem.at[b])

    def _scatter(c, b):
        return pltpu.async_copy(rows_v.at[b], agg_sh.at[dstv.at[c]],
                                ssem.at[b], add=True)

    for b in range(NB):
        _gather(b, b)

    @pl.loop(0, K - NB, step=NB)
    def _(j):
        for b in range(NB):
            pltpu.make_async_copy(y_hbm.at[srcv.at[j + b]], rows_v.at[b],
                                  gsem.at[b]).wait()
            _scatter(j + b, b)
        for b in range(NB):
            pltpu.make_async_copy(rows_v.at[b], agg_sh.at[dstv.at[j + b]],
                                  ssem.at[b]).wait()
            _gather(j + NB + b, b)

    for b in range(NB):
        c = K - NB + b
        pltpu.make_async_copy(y_hbm.at[srcv.at[c]], rows_v.at[b],
                              gsem.at[b]).wait()
        _scatter(c, b)
    for b in range(NB):
        c = K - NB + b
        pltpu.make_async_copy(rows_v.at[b], agg_sh.at[dstv.at[c]],
                              ssem.at[b]).wait()

    plsc.subcore_barrier()

    for p in range(NZP):
        sl = pl.ds(sid * ROWS_PER_SUB + p * CHUNK, CHUNK)
        pltpu.sync_copy(agg_sh.at[sl], rows_v.at[p])
        pltpu.sync_copy(rows_v.at[p], out_hbm.at[cid].at[sl])


def _agg_partials(src_r, dst_r, y):
    return pl.kernel(
        _agg_body,
        out_type=jax.ShapeDtypeStruct((NSC, N_PAD, H1), jnp.float32),
        mesh=_mesh,
        scratch_types=[
            pltpu.VMEM((K, CHUNK), jnp.int32),
            pltpu.VMEM((K, CHUNK), jnp.int32),
            pltpu.VMEM((NB, CHUNK, H1), jnp.float32),
            pltpu.VMEM_SHARED((N_PAD, H1), jnp.float32),
            pltpu.SemaphoreType.DMA((NB,)),
            pltpu.SemaphoreType.DMA((NB,)),
        ],
        compiler_params=_sc_params,
    )(src_r, dst_r, y)


def _xw_scale_body(x_ref, w_ref, dp_ref, y_ref, dinv_ref):
    xw = jnp.dot(x_ref[...], w_ref[...], preferred_element_type=jnp.float32)
    deg = dp_ref[0] + dp_ref[1] + 1.0
    dinv = lax.rsqrt(deg)
    dinv_ref[...] = dinv
    y_ref[...] = xw * dinv


def _mlp_body(ap_ref, y_ref, dinv_ref, b1_ref, lw1_ref, lb1_ref,
              lw2_ref, lb2_ref, lw3_ref, lb3_ref, o_ref):
    agg = ap_ref[0] + ap_ref[1] + y_ref[...]
    h = jnp.maximum(agg * dinv_ref[...] + b1_ref[...], 0.0)
    h = jnp.maximum(jnp.dot(h, lw1_ref[...],
                            preferred_element_type=jnp.float32) + lb1_ref[...], 0.0)
    h = jnp.maximum(jnp.dot(h, lw2_ref[...],
                            preferred_element_type=jnp.float32) + lb2_ref[...], 0.0)
    o_ref[...] = jnp.dot(h, lw3_ref[...],
                         preferred_element_type=jnp.float32) + lb3_ref[...]


def kernel(x, edge_index, W1, b1, lW1, lb1, lW2, lb2, lW3, lb3):
    f32 = jnp.float32
    src = edge_index[0]
    dst = edge_index[1]
    pad = E_PAD - E
    src_r = jnp.concatenate(
        [src, jnp.full((pad,), N, jnp.int32)]).reshape(NW, K, CHUNK)
    dst_r = jnp.concatenate(
        [dst, jnp.full((pad,), N, jnp.int32)]).reshape(NW, K, CHUNK)
    x_pad = jnp.pad(x, ((0, N_PAD - N), (0, 0)))

    deg_p = _deg_partials(dst_r)
    dp = deg_p[:, :, 0:1]

    y, dinv = pl.pallas_call(
        _xw_scale_body,
        out_shape=[jax.ShapeDtypeStruct((N_PAD, H1), f32),
                   jax.ShapeDtypeStruct((N_PAD, 1), f32)],
    )(x_pad, W1, dp)

    agg_p = _agg_partials(src_r, dst_r, y)

    out = pl.pallas_call(
        _mlp_body,
        out_shape=jax.ShapeDtypeStruct((N_PAD, NCLS), f32),
    )(agg_p, y, dinv, b1.reshape(1, H1), lW1, lb1.reshape(1, H2),
      lW2, lb2.reshape(1, H3), lW3, lb3.reshape(1, NCLS))

    return out[:N]


# speedup vs baseline: 40.2841x; 1.4888x over previous
import functools

import jax
import jax.numpy as jnp
from jax import lax
from jax.experimental import pallas as pl
from jax.experimental.pallas import tpu as pltpu
from jax.experimental.pallas import tpu_sc as plsc

N = 10000
E = 320000
D = 128
H1, H2, H3 = 64, 32, 16
NCLS = 10

NSC = 2
NSUB = 16
NW = NSC * NSUB
CHUNK = 128
K = 80
E_PAD = NW * K * CHUNK
N_PAD = 10240
ROWS_PER_SUB = N_PAD // NSUB
DEG_W = 16

_mesh = plsc.VectorSubcoreMesh(core_axis_name="c", subcore_axis_name="s")
_sc_params = pltpu.CompilerParams(use_tc_tiling_on_sc=False)


def _deg_body(dst_hbm, out_hbm, idx_v, val_v, zbuf_v, deg_sh, sem0):
    cid = lax.axis_index("c")
    sid = lax.axis_index("s")
    wid = sid * NSC + cid

    @pl.loop(0, ROWS_PER_SUB)
    def _(r):
        zbuf_v[r, :] = jnp.zeros((DEG_W,), jnp.float32)

    pltpu.sync_copy(zbuf_v, deg_sh.at[pl.ds(sid * ROWS_PER_SUB, ROWS_PER_SUB)])

    @pl.loop(0, CHUNK)
    def _(r):
        val_v[r, :] = jnp.ones((DEG_W,), jnp.float32)

    pltpu.sync_copy(dst_hbm.at[wid], idx_v)
    plsc.subcore_barrier()

    @pl.loop(0, K, step=8)
    def _(j):
        cps = [pltpu.async_copy(val_v, deg_sh.at[idx_v.at[j + b]], sem0,
                                add=True)
               for b in range(8)]
        for cp in cps:
            cp.wait()

    plsc.subcore_barrier()

    sl = pl.ds(sid * ROWS_PER_SUB, ROWS_PER_SUB)
    pltpu.sync_copy(deg_sh.at[sl], zbuf_v)
    pltpu.sync_copy(zbuf_v, out_hbm.at[cid].at[sl])


def _deg_partials(dst_r):
    return pl.kernel(
        _deg_body,
        out_type=jax.ShapeDtypeStruct((NSC, N_PAD, DEG_W), jnp.float32),
        mesh=_mesh,
        scratch_types=[
            pltpu.VMEM((K, CHUNK), jnp.int32),
            pltpu.VMEM((CHUNK, DEG_W), jnp.float32),
            pltpu.VMEM((ROWS_PER_SUB, DEG_W), jnp.float32),
            pltpu.VMEM_SHARED((N_PAD, DEG_W), jnp.float32),
            pltpu.SemaphoreType.DMA,
        ],
        compiler_params=_sc_params,
    )(dst_r)


NB = 4
NZP = ROWS_PER_SUB // CHUNK
NQ = 4
KQ = K // NQ


def _agg_body(src_hbm, dst_hbm, y_hbm, out_hbm,
              srcv, dstv, rows_v, agg_sh, y_sh, gsem, ssem):
    cid = lax.axis_index("c")
    sid = lax.axis_index("s")
    wid = sid * NSC + cid

    @pl.loop(0, CHUNK)
    def _(r):
        for h in range(H1 // 16):
            rows_v[0, r, pl.ds(h * 16, 16)] = jnp.zeros((16,), jnp.float32)

    for p in range(NZP):
        pltpu.sync_copy(
            rows_v.at[0],
            agg_sh.at[pl.ds(sid * ROWS_PER_SUB + p * CHUNK, CHUNK)])

    ysl = pl.ds(sid * ROWS_PER_SUB, ROWS_PER_SUB)
    pltpu.sync_copy(y_hbm.at[ysl], y_sh.at[ysl])
    plsc.subcore_barrier()

    def _gather(c, b):
        return pltpu.async_copy(y_sh.at[srcv.at[c]], rows_v.at[b],
                                gsem.at[b])

    def _scatter(c, b):
        return pltpu.async_copy(rows_v.at[b], agg_sh.at[dstv.at[c]],
                                ssem.at[b], add=True)

    @pl.loop(0, NQ)
    def _(q):
        qsl = pl.ds(q * KQ, KQ)
        pltpu.sync_copy(src_hbm.at[wid].at[qsl], srcv)
        pltpu.sync_copy(dst_hbm.at[wid].at[qsl], dstv)

        for b in range(NB):
            _gather(b, b)

        @pl.loop(0, KQ - NB, step=NB)
        def _(j):
            for b in range(NB):
                pltpu.make_async_copy(y_sh.at[srcv.at[j + b]], rows_v.at[b],
                                      gsem.at[b]).wait()
                _scatter(j + b, b)
            for b in range(NB):
                pltpu.make_async_copy(rows_v.at[b], agg_sh.at[dstv.at[j + b]],
                                      ssem.at[b]).wait()
                _gather(j + NB + b, b)

        for b in range(NB):
            c = KQ - NB + b
            pltpu.make_async_copy(y_sh.at[srcv.at[c]], rows_v.at[b],
                                  gsem.at[b]).wait()
            _scatter(c, b)
        for b in range(NB):
            c = KQ - NB + b
            pltpu.make_async_copy(rows_v.at[b], agg_sh.at[dstv.at[c]],
                                  ssem.at[b]).wait()

    plsc.subcore_barrier()

    for p in range(NZP):
        sl = pl.ds(sid * ROWS_PER_SUB + p * CHUNK, CHUNK)
        pltpu.sync_copy(agg_sh.at[sl], rows_v.at[p % NB])
        pltpu.sync_copy(rows_v.at[p % NB], out_hbm.at[cid].at[sl])


def _agg_partials(src_r, dst_r, y):
    return pl.kernel(
        _agg_body,
        out_type=jax.ShapeDtypeStruct((NSC, N_PAD, H1), jnp.float32),
        mesh=_mesh,
        scratch_types=[
            pltpu.VMEM((KQ, CHUNK), jnp.int32),
            pltpu.VMEM((KQ, CHUNK), jnp.int32),
            pltpu.VMEM((NB, CHUNK, H1), jnp.float32),
            pltpu.VMEM_SHARED((N_PAD, H1), jnp.float32),
            pltpu.VMEM_SHARED((N_PAD, H1), jnp.float32),
            pltpu.SemaphoreType.DMA((NB,)),
            pltpu.SemaphoreType.DMA((NB,)),
        ],
        compiler_params=_sc_params,
    )(src_r, dst_r, y)


def _xw_scale_body(x_ref, w_ref, dp_ref, y_ref, dinv_ref):
    xw = jnp.dot(x_ref[...], w_ref[...], preferred_element_type=jnp.float32)
    deg = dp_ref[0] + dp_ref[1] + 1.0
    dinv = lax.rsqrt(deg)
    dinv_ref[...] = dinv
    y_ref[...] = xw * dinv


def _mlp_body(ap_ref, y_ref, dinv_ref, b1_ref, lw1_ref, lb1_ref,
              lw2_ref, lb2_ref, lw3_ref, lb3_ref, o_ref):
    agg = ap_ref[0] + ap_ref[1] + y_ref[...]
    h = jnp.maximum(agg * dinv_ref[...] + b1_ref[...], 0.0)
    h = jnp.maximum(jnp.dot(h, lw1_ref[...],
                            preferred_element_type=jnp.float32) + lb1_ref[...], 0.0)
    h = jnp.maximum(jnp.dot(h, lw2_ref[...],
                            preferred_element_type=jnp.float32) + lb2_ref[...], 0.0)
    o_ref[...] = jnp.dot(h, lw3_ref[...],
                         preferred_element_type=jnp.float32) + lb3_ref[...]


def kernel(x, edge_index, W1, b1, lW1, lb1, lW2, lb2, lW3, lb3):
    f32 = jnp.float32
    src = edge_index[0]
    dst = edge_index[1]
    pad = E_PAD - E
    src_r = jnp.concatenate(
        [src, jnp.full((pad,), N, jnp.int32)]).reshape(NW, K, CHUNK)
    dst_r = jnp.concatenate(
        [dst, jnp.full((pad,), N, jnp.int32)]).reshape(NW, K, CHUNK)
    x_pad = jnp.pad(x, ((0, N_PAD - N), (0, 0)))

    deg_p = _deg_partials(dst_r)
    dp = deg_p[:, :, 0:1]

    y, dinv = pl.pallas_call(
        _xw_scale_body,
        out_shape=[jax.ShapeDtypeStruct((N_PAD, H1), f32),
                   jax.ShapeDtypeStruct((N_PAD, 1), f32)],
    )(x_pad, W1, dp)

    agg_p = _agg_partials(src_r, dst_r, y)

    out = pl.pallas_call(
        _mlp_body,
        out_shape=jax.ShapeDtypeStruct((N_PAD, NCLS), f32),
    )(agg_p, y, dinv, b1.reshape(1, H1), lW1, lb1.reshape(1, H2),
      lW2, lb2.reshape(1, H3), lW3, lb3.reshape(1, NCLS))

    return out[:N]
